# baseline (speedup 1.0000x reference)
import functools

import jax
import jax.numpy as jnp
from jax import lax
from jax.experimental import pallas as pl

B = 1024
D = 768
FF = 3072
E = 8
L = 6
CAP = 160

_F32 = jnp.float32
_INTERPRET = False


def _rms(h, w):
    return h * lax.rsqrt(jnp.mean(h * h, axis=-1, keepdims=True) + 1e-6) * w


def _dot(a, b):
    return jnp.dot(a, b, preferred_element_type=_F32)


def _proj_body(x_ref, w_ref, b_ref, out_ref):
    out_ref[...] = _dot(x_ref[...], w_ref[...]) + b_ref[0, :]


def _proj(x2, w, b2):
    return pl.pallas_call(
        _proj_body,
        grid=(4,),
        in_specs=[
            pl.BlockSpec((B // 4, 3072), lambda i: (i, 0)),
            pl.BlockSpec((3072, D), lambda i: (0, 0)),
            pl.BlockSpec((1, D), lambda i: (0, 0)),
        ],
        out_specs=pl.BlockSpec((B // 4, D), lambda i: (i, 0)),
        out_shape=jax.ShapeDtypeStruct((B, D), _F32),
        interpret=_INTERPRET,
    )(x2, w, b2)


def _att_body(h_ref, ln1_ref, wv_ref, wo_ref, out_ref):
    h = h_ref[...]
    n = _rms(h, ln1_ref[0, :])
    out_ref[...] = h + _dot(_dot(n, wv_ref[...]), wo_ref[...])


def _att(h, ln1, wv, wo):
    return pl.pallas_call(
        _att_body,
        grid=(4,),
        in_specs=[
            pl.BlockSpec((B // 4, D), lambda i: (i, 0)),
            pl.BlockSpec((1, D), lambda i: (0, 0)),
            pl.BlockSpec((D, D), lambda i: (0, 0)),
            pl.BlockSpec((D, D), lambda i: (0, 0)),
        ],
        out_specs=pl.BlockSpec((B // 4, D), lambda i: (i, 0)),
        out_shape=jax.ShapeDtypeStruct((B, D), _F32),
        interpret=_INTERPRET,
    )(h, ln1, wv, wo)


def _att_route_body(h_ref, ln1_ref, wv_ref, wo_ref, ln2_ref, rw_ref,
                    hout_ref, n2_ref, log_ref):
    h = h_ref[...]
    n = _rms(h, ln1_ref[0, :])
    hn = h + _dot(_dot(n, wv_ref[...]), wo_ref[...])
    hout_ref[...] = hn
    n2 = _rms(hn, ln2_ref[0, :])
    n2_ref[...] = n2
    log_ref[...] = _dot(n2, rw_ref[...])


def _att_route(h, ln1, wv, wo, ln2, rw_pad):
    return pl.pallas_call(
        _att_route_body,
        grid=(4,),
        in_specs=[
            pl.BlockSpec((B // 4, D), lambda i: (i, 0)),
            pl.BlockSpec((1, D), lambda i: (0, 0)),
            pl.BlockSpec((D, D), lambda i: (0, 0)),
            pl.BlockSpec((D, D), lambda i: (0, 0)),
            pl.BlockSpec((1, D), lambda i: (0, 0)),
            pl.BlockSpec((D, 128), lambda i: (0, 0)),
        ],
        out_specs=[
            pl.BlockSpec((B // 4, D), lambda i: (i, 0)),
            pl.BlockSpec((B // 4, D), lambda i: (i, 0)),
            pl.BlockSpec((B // 4, 128), lambda i: (i, 0)),
        ],
        out_shape=[
            jax.ShapeDtypeStruct((B, D), _F32),
            jax.ShapeDtypeStruct((B, D), _F32),
            jax.ShapeDtypeStruct((B, 128), _F32),
        ],
        interpret=_INTERPRET,
    )(h, ln1, wv, wo, ln2, rw_pad)


def _route_body(log_ref, slot_ref, gate_ref):
    l = log_ref[...]
    lane = lax.broadcasted_iota(jnp.int32, (B, 128), 1)
    valid = lane < E
    lm = jnp.where(valid, l, -1e30)
    m = jnp.max(lm, axis=-1, keepdims=True)
    p = jnp.where(valid, jnp.exp(lm - m), 0.0)
    probs = p / jnp.sum(p, axis=-1, keepdims=True)
    gate = jnp.max(probs, axis=-1, keepdims=True)
    hit = (probs == gate) & valid
    idx = jnp.min(jnp.where(hit, lane, 10**6), axis=-1, keepdims=True)
    onehot = jnp.where(lane == idx, 1.0, 0.0).astype(_F32)
    r = lax.broadcasted_iota(jnp.int32, (B, B), 0)
    c = lax.broadcasted_iota(jnp.int32, (B, B), 1)
    tri = (c <= r).astype(_F32)
    counts = _dot(tri, onehot)
    pos = jnp.sum(jnp.where(lane == idx, counts, 0.0), axis=-1,
                  keepdims=True) - 1.0
    kept = pos < float(CAP)
    slot = jnp.where(kept, idx.astype(_F32) * CAP + pos, 1e7)
    slot_ref[...] = jnp.broadcast_to(slot, (B, 128))
    gate_ref[...] = jnp.broadcast_to(gate, (B, 128))


def _route(logits):
    return pl.pallas_call(
        _route_body,
        out_shape=[
            jax.ShapeDtypeStruct((B, 128), _F32),
            jax.ShapeDtypeStruct((B, 128), _F32),
        ],
        interpret=_INTERPRET,
    )(logits)


def _dispatch_body(slot_ref, n2_ref, ein_ref):
    s0 = pl.program_id(0) * CAP
    slot = slot_ref[...][:, 0:1].astype(jnp.int32)
    lane = lax.broadcasted_iota(jnp.int32, (B, CAP), 1) + s0
    mt = (jnp.broadcast_to(slot, (B, CAP)) == lane).astype(_F32)
    ein_ref[0] = lax.dot_general(
        n2_ref[...], mt, (((0,), (0,)), ((), ())),
        preferred_element_type=_F32)


def _dispatch(slot_b, n2):
    return pl.pallas_call(
        _dispatch_body,
        grid=(E,),
        in_specs=[
            pl.BlockSpec((B, 128), lambda i: (0, 0)),
            pl.BlockSpec((B, D), lambda i: (0, 0)),
        ],
        out_specs=pl.BlockSpec((1, D, CAP), lambda i: (i, 0, 0)),
        out_shape=jax.ShapeDtypeStruct((E, D, CAP), _F32),
        interpret=_INTERPRET,
    )(slot_b, n2)


def _expert_body(eint_ref, wi_ref, wo_ref, out_ref):
    eint = eint_ref[0]
    h1_t = jnp.maximum(lax.dot_general(
        wi_ref[0], eint, (((0,), (0,)), ((), ())),
        preferred_element_type=_F32), 0.0)
    out_ref[0] = lax.dot_general(
        wo_ref[0], h1_t, (((0,), (0,)), ((), ())),
        preferred_element_type=_F32)


def _experts(ein_t, wi, wo):
    return pl.pallas_call(
        _expert_body,
        grid=(E,),
        in_specs=[
            pl.BlockSpec((1, D, CAP), lambda e: (e, 0, 0)),
            pl.BlockSpec((1, D, FF), lambda e: (e, 0, 0)),
            pl.BlockSpec((1, FF, D), lambda e: (e, 0, 0)),
        ],
        out_specs=pl.BlockSpec((1, D, CAP), lambda e: (e, 0, 0)),
        out_shape=jax.ShapeDtypeStruct((E, D, CAP), _F32),
        interpret=_INTERPRET,
    )(ein_t, wi, wo)


def _combine_body(slot_ref, gate_ref, h2_ref, h_ref, out_ref):
    slot = slot_ref[...][:, 0:1].astype(jnp.int32)
    lane = lax.broadcasted_iota(jnp.int32, (256, CAP), 1)
    gate = gate_ref[...][:, 0:1]
    y = None
    for e in range(E):
        ne = (jnp.broadcast_to(slot - e * CAP, (256, CAP))
              == lane).astype(_F32)
        part = lax.dot_general(ne, h2_ref[e], (((1,), (1,)), ((), ())),
                               preferred_element_type=_F32)
        y = part if y is None else y + part
    out_ref[...] = h_ref[...] + jnp.broadcast_to(gate, (256, D)) * y


def _combine(slot_b, gate_b, h2_t, h):
    return pl.pallas_call(
        _combine_body,
        grid=(4,),
        in_specs=[
            pl.BlockSpec((256, 128), lambda i: (i, 0)),
            pl.BlockSpec((256, 128), lambda i: (i, 0)),
            pl.BlockSpec((E, D, CAP), lambda i: (0, 0, 0)),
            pl.BlockSpec((256, D), lambda i: (i, 0)),
        ],
        out_specs=pl.BlockSpec((256, D), lambda i: (i, 0)),
        out_shape=jax.ShapeDtypeStruct((B, D), _F32),
        interpret=_INTERPRET,
    )(slot_b, gate_b, h2_t, h)


def _ffn_body(h_ref, ln2_ref, wi_ref, wo_ref, out_ref):
    f = pl.program_id(1)
    h = h_ref[...]
    n2 = _rms(h, ln2_ref[0, :])
    h1 = jnp.maximum(_dot(n2, wi_ref[...]), 0.0)
    part = _dot(h1, wo_ref[...])

    @pl.when(f == 0)
    def _():
        out_ref[...] = h + part

    @pl.when(f != 0)
    def _():
        out_ref[...] += part


def _ffn(h, ln2, wi, wo):
    fb = FF // 2
    return pl.pallas_call(
        _ffn_body,
        grid=(4, 2),
        in_specs=[
            pl.BlockSpec((B // 4, D), lambda i, f: (i, 0)),
            pl.BlockSpec((1, D), lambda i, f: (0, 0)),
            pl.BlockSpec((D, fb), lambda i, f: (0, f)),
            pl.BlockSpec((fb, D), lambda i, f: (f, 0)),
        ],
        out_specs=pl.BlockSpec((B // 4, D), lambda i, f: (i, 0)),
        out_shape=jax.ShapeDtypeStruct((B, D), _F32),
        interpret=_INTERPRET,
    )(h, ln2, wi, wo)


def _final_body(h_ref, ln_ref, w_ref, b_ref, out_ref):
    n = _rms(h_ref[...], ln_ref[0, :])
    out_ref[...] = _dot(n, w_ref[...]) + b_ref[0, :]


def _final(h, final_ln, fcw_pad, fcb_pad):
    return pl.pallas_call(
        _final_body,
        out_shape=jax.ShapeDtypeStruct((B, 128), _F32),
        interpret=_INTERPRET,
    )(h, final_ln, fcw_pad, fcb_pad)


def kernel(x, proj_W, proj_b, attn_q, attn_k, attn_v, attn_o, ln1, ln2,
           router_W, moe_wi, moe_wo, ffn_wi, ffn_wo, final_ln, fc_W, fc_b):
    del attn_q, attn_k
    x2 = x.reshape(B, -1)
    h = _proj(x2, proj_W, proj_b.reshape(1, D))
    for i in range(L):
        j = i // 2
        if i % 2 == 0:
            h = _att(h, ln1[i].reshape(1, D), attn_v[i], attn_o[i])
            h = _ffn(h, ln2[i].reshape(1, D), ffn_wi[j], ffn_wo[j])
        else:
            rw_pad = jnp.pad(router_W[j], ((0, 0), (0, 128 - E)))
            h, n2, logits = _att_route(
                h, ln1[i].reshape(1, D), attn_v[i], attn_o[i],
                ln2[i].reshape(1, D), rw_pad)
            slot_b, gate_b = _route(logits)
            ein = _dispatch(slot_b, n2)
            h2 = _experts(ein, moe_wi[j], moe_wo[j])
            h = _combine(slot_b, gate_b, h2, h)
    out = _final(h, final_ln.reshape(1, D),
                 jnp.pad(fc_W, ((0, 0), (0, 128 - 10))),
                 jnp.pad(fc_b, (0, 128 - 10)).reshape(1, 128))
    return out[:, :10]


# speedup vs baseline: 1.8230x; 1.8230x over previous
import jax
import jax.numpy as jnp
from jax import lax
from jax.experimental import pallas as pl

B = 1024
D = 768
FF = 3072
E = 8
L = 6
CAP = 160

_F32 = jnp.float32
_INTERPRET = False


def _rms(h, w):
    return h * lax.rsqrt(jnp.mean(h * h, axis=-1, keepdims=True) + 1e-6) * w


def _dot(a, b):
    return jnp.dot(a, b, preferred_element_type=_F32)


def _proj_body(x_ref, w_ref, b_ref, out_ref):
    out_ref[...] = _dot(x_ref[...], w_ref[...]) + b_ref[0, :]


def _proj(x2, w, b):
    return pl.pallas_call(
        _proj_body,
        grid=(4,),
        in_specs=[
            pl.BlockSpec((B // 4, 3072), lambda i: (i, 0)),
            pl.BlockSpec((3072, D), lambda i: (0, 0)),
            pl.BlockSpec((1, D), lambda i: (0, 0)),
        ],
        out_specs=pl.BlockSpec((B // 4, D), lambda i: (i, 0)),
        out_shape=jax.ShapeDtypeStruct((B, D), _F32),
        interpret=_INTERPRET,
    )(x2, w, b.reshape(1, D))


def _att_body(h_ref, ln1_ref, wv_ref, wo_ref, out_ref):
    h = h_ref[...]
    n = _rms(h, ln1_ref[0, 0, :])
    out_ref[...] = h + _dot(_dot(n, wv_ref[0]), wo_ref[0])


def _att(h, ln1, wv, wo, i):
    return pl.pallas_call(
        _att_body,
        grid=(4,),
        in_specs=[
            pl.BlockSpec((B // 4, D), lambda r: (r, 0)),
            pl.BlockSpec((1, 1, D), lambda r, i=i: (i, 0, 0)),
            pl.BlockSpec((1, D, D), lambda r, i=i: (i, 0, 0)),
            pl.BlockSpec((1, D, D), lambda r, i=i: (i, 0, 0)),
        ],
        out_specs=pl.BlockSpec((B // 4, D), lambda r: (r, 0)),
        out_shape=jax.ShapeDtypeStruct((B, D), _F32),
        interpret=_INTERPRET,
    )(h, ln1, wv, wo)


def _att_route_body(h_ref, ln1_ref, wv_ref, wo_ref, ln2_ref, rw_ref,
                    hout_ref, n2_ref, log_ref):
    h = h_ref[...]
    n = _rms(h, ln1_ref[0, 0, :])
    hn = h + _dot(_dot(n, wv_ref[0]), wo_ref[0])
    hout_ref[...] = hn
    n2 = _rms(hn, ln2_ref[0, 0, :])
    n2_ref[...] = n2
    log_ref[...] = _dot(n2, rw_ref[...])


def _att_route(h, ln1, wv, wo, ln2, rw_pad, i):
    return pl.pallas_call(
        _att_route_body,
        grid=(4,),
        in_specs=[
            pl.BlockSpec((B // 4, D), lambda r: (r, 0)),
            pl.BlockSpec((1, 1, D), lambda r, i=i: (i, 0, 0)),
            pl.BlockSpec((1, D, D), lambda r, i=i: (i, 0, 0)),
            pl.BlockSpec((1, D, D), lambda r, i=i: (i, 0, 0)),
            pl.BlockSpec((1, 1, D), lambda r, i=i: (i, 0, 0)),
            pl.BlockSpec((D, 128), lambda r: (0, 0)),
        ],
        out_specs=[
            pl.BlockSpec((B // 4, D), lambda r: (r, 0)),
            pl.BlockSpec((B // 4, D), lambda r: (r, 0)),
            pl.BlockSpec((B // 4, 128), lambda r: (r, 0)),
        ],
        out_shape=[
            jax.ShapeDtypeStruct((B, D), _F32),
            jax.ShapeDtypeStruct((B, D), _F32),
            jax.ShapeDtypeStruct((B, 128), _F32),
        ],
        interpret=_INTERPRET,
    )(h, ln1, wv, wo, ln2, rw_pad)


def _route_body(log_ref, slot_ref, gate_ref):
    l = log_ref[...]
    lane = lax.broadcasted_iota(jnp.int32, (B, 128), 1)
    valid = lane < E
    lm = jnp.where(valid, l, -1e30)
    m = jnp.max(lm, axis=-1, keepdims=True)
    p = jnp.where(valid, jnp.exp(lm - m), 0.0)
    probs = p / jnp.sum(p, axis=-1, keepdims=True)
    gate = jnp.max(probs, axis=-1, keepdims=True)
    hit = (probs == gate) & valid
    idx = jnp.min(jnp.where(hit, lane, 10**6), axis=-1, keepdims=True)
    onehot = jnp.where(lane == idx, 1.0, 0.0).astype(_F32)
    r = lax.broadcasted_iota(jnp.int32, (B, B), 0)
    c = lax.broadcasted_iota(jnp.int32, (B, B), 1)
    tri = (c <= r).astype(_F32)
    counts = _dot(tri, onehot)
    pos = jnp.sum(jnp.where(lane == idx, counts, 0.0), axis=-1,
                  keepdims=True) - 1.0
    kept = pos < float(CAP)
    slot = jnp.where(kept, idx.astype(_F32) * CAP + pos, 1e7)
    slot_ref[...] = jnp.broadcast_to(slot, (B, 128))
    gate_ref[...] = jnp.broadcast_to(gate, (B, 128))


def _route(logits):
    return pl.pallas_call(
        _route_body,
        out_shape=[
            jax.ShapeDtypeStruct((B, 128), _F32),
            jax.ShapeDtypeStruct((B, 128), _F32),
        ],
        interpret=_INTERPRET,
    )(logits)


def _dispatch_body(slot_ref, n2_ref, ein_ref):
    s0 = pl.program_id(0) * CAP
    slot = slot_ref[...][:, 0:1].astype(jnp.int32)
    lane = lax.broadcasted_iota(jnp.int32, (B, CAP), 1) + s0
    mt = (jnp.broadcast_to(slot, (B, CAP)) == lane).astype(_F32)
    ein_ref[0] = lax.dot_general(
        n2_ref[...], mt, (((0,), (0,)), ((), ())),
        preferred_element_type=_F32)


def _dispatch(slot_b, n2):
    return pl.pallas_call(
        _dispatch_body,
        grid=(E,),
        in_specs=[
            pl.BlockSpec((B, 128), lambda i: (0, 0)),
            pl.BlockSpec((B, D), lambda i: (0, 0)),
        ],
        out_specs=pl.BlockSpec((1, D, CAP), lambda i: (i, 0, 0)),
        out_shape=jax.ShapeDtypeStruct((E, D, CAP), _F32),
        interpret=_INTERPRET,
    )(slot_b, n2)


def _expert_body(eint_ref, wi_ref, wo_ref, out_ref):
    eint = eint_ref[0]
    h1_t = jnp.maximum(lax.dot_general(
        wi_ref[0, 0], eint, (((0,), (0,)), ((), ())),
        preferred_element_type=_F32), 0.0)
    out_ref[0] = lax.dot_general(
        wo_ref[0, 0], h1_t, (((0,), (0,)), ((), ())),
        preferred_element_type=_F32)


def _experts(ein_t, wi, wo, j):
    return pl.pallas_call(
        _expert_body,
        grid=(E,),
        in_specs=[
            pl.BlockSpec((1, D, CAP), lambda e: (e, 0, 0)),
            pl.BlockSpec((1, 1, D, FF), lambda e, j=j: (j, e, 0, 0)),
            pl.BlockSpec((1, 1, FF, D), lambda e, j=j: (j, e, 0, 0)),
        ],
        out_specs=pl.BlockSpec((1, D, CAP), lambda e: (e, 0, 0)),
        out_shape=jax.ShapeDtypeStruct((E, D, CAP), _F32),
        interpret=_INTERPRET,
    )(ein_t, wi, wo)


def _combine_body(slot_ref, gate_ref, h2_ref, h_ref, out_ref):
    slot = slot_ref[...][:, 0:1].astype(jnp.int32)
    lane = lax.broadcasted_iota(jnp.int32, (256, CAP), 1)
    gate = gate_ref[...][:, 0:1]
    y = None
    for e in range(E):
        ne = (jnp.broadcast_to(slot - e * CAP, (256, CAP))
              == lane).astype(_F32)
        part = lax.dot_general(ne, h2_ref[e], (((1,), (1,)), ((), ())),
                               preferred_element_type=_F32)
        y = part if y is None else y + part
    out_ref[...] = h_ref[...] + jnp.broadcast_to(gate, (256, D)) * y


def _combine(slot_b, gate_b, h2_t, h):
    return pl.pallas_call(
        _combine_body,
        grid=(4,),
        in_specs=[
            pl.BlockSpec((256, 128), lambda i: (i, 0)),
            pl.BlockSpec((256, 128), lambda i: (i, 0)),
            pl.BlockSpec((E, D, CAP), lambda i: (0, 0, 0)),
            pl.BlockSpec((256, D), lambda i: (i, 0)),
        ],
        out_specs=pl.BlockSpec((256, D), lambda i: (i, 0)),
        out_shape=jax.ShapeDtypeStruct((B, D), _F32),
        interpret=_INTERPRET,
    )(slot_b, gate_b, h2_t, h)


def _ffn_body(h_ref, ln2_ref, wi_ref, wo_ref, out_ref):
    f = pl.program_id(1)
    h = h_ref[...]
    n2 = _rms(h, ln2_ref[0, 0, :])
    h1 = jnp.maximum(_dot(n2, wi_ref[0]), 0.0)
    part = _dot(h1, wo_ref[0])

    @pl.when(f == 0)
    def _():
        out_ref[...] = h + part

    @pl.when(f != 0)
    def _():
        out_ref[...] += part


def _ffn(h, ln2, wi, wo, i, j):
    fb = FF // 2
    return pl.pallas_call(
        _ffn_body,
        grid=(4, 2),
        in_specs=[
            pl.BlockSpec((B // 4, D), lambda r, f: (r, 0)),
            pl.BlockSpec((1, 1, D), lambda r, f, i=i: (i, 0, 0)),
            pl.BlockSpec((1, D, fb), lambda r, f, j=j: (j, 0, f)),
            pl.BlockSpec((1, fb, D), lambda r, f, j=j: (j, f, 0)),
        ],
        out_specs=pl.BlockSpec((B // 4, D), lambda r, f: (r, 0)),
        out_shape=jax.ShapeDtypeStruct((B, D), _F32),
        interpret=_INTERPRET,
    )(h, ln2, wi, wo)


def _final_body(h_ref, ln_ref, w_ref, b_ref, out_ref):
    n = _rms(h_ref[...], ln_ref[0, :])
    out_ref[...] = _dot(n, w_ref[...]) + b_ref[0, :]


def _final(h, final_ln, fcw_pad, fcb_pad):
    return pl.pallas_call(
        _final_body,
        out_shape=jax.ShapeDtypeStruct((B, 128), _F32),
        interpret=_INTERPRET,
    )(h, final_ln, fcw_pad, fcb_pad)


def kernel(x, proj_W, proj_b, attn_q, attn_k, attn_v, attn_o, ln1, ln2,
           router_W, moe_wi, moe_wo, ffn_wi, ffn_wo, final_ln, fc_W, fc_b):
    del attn_q, attn_k
    x2 = x.reshape(B, -1)
    ln1 = ln1.reshape(L, 1, D)
    ln2 = ln2.reshape(L, 1, D)
    h = _proj(x2, proj_W, proj_b)
    rw_pad = jnp.pad(router_W, ((0, 0), (0, 0), (0, 128 - E)))
    for i in range(L):
        j = i // 2
        if i % 2 == 0:
            h = _att(h, ln1, attn_v, attn_o, i)
            h = _ffn(h, ln2, ffn_wi, ffn_wo, i, j)
        else:
            h, n2, logits = _att_route(h, ln1, attn_v, attn_o, ln2,
                                       rw_pad[j], i)
            slot_b, gate_b = _route(logits)
            ein_t = _dispatch(slot_b, n2)
            h2_t = _experts(ein_t, moe_wi, moe_wo, j)
            h = _combine(slot_b, gate_b, h2_t, h)
    out = _final(h, final_ln.reshape(1, D),
                 jnp.pad(fc_W, ((0, 0), (0, 128 - 10))),
                 jnp.pad(fc_b, (0, 128 - 10)).reshape(1, 128))
    return out[:, :10]


# speedup vs baseline: 2.0037x; 1.0991x over previous
import jax
import jax.numpy as jnp
from jax import lax
from jax.experimental import pallas as pl

B = 1024
D = 768
FF = 3072
E = 8
L = 6
CAP = 160

_F32 = jnp.float32
_INTERPRET = False


def _rms(h, w):
    return h * lax.rsqrt(jnp.mean(h * h, axis=-1, keepdims=True) + 1e-6) * w


def _dot(a, b):
    return jnp.dot(a, b, preferred_element_type=_F32)


def _proj_body(x_ref, w_ref, b_ref, out_ref):
    out_ref[...] = _dot(x_ref[...], w_ref[...]) + b_ref[0, :]


def _proj(x2, w, b):
    return pl.pallas_call(
        _proj_body,
        grid=(4,),
        in_specs=[
            pl.BlockSpec((B // 4, 3072), lambda i: (i, 0)),
            pl.BlockSpec((3072, D), lambda i: (0, 0)),
            pl.BlockSpec((1, D), lambda i: (0, 0)),
        ],
        out_specs=pl.BlockSpec((B // 4, D), lambda i: (i, 0)),
        out_shape=jax.ShapeDtypeStruct((B, D), _F32),
        interpret=_INTERPRET,
    )(x2, w, b.reshape(1, D))


def _att_body(h_ref, ln1_ref, wv_ref, wo_ref, out_ref):
    h = h_ref[...]
    n = _rms(h, ln1_ref[0, 0, :])
    out_ref[...] = h + _dot(_dot(n, wv_ref[0]), wo_ref[0])


def _att(h, ln1, wv, wo, i):
    return pl.pallas_call(
        _att_body,
        grid=(4,),
        in_specs=[
            pl.BlockSpec((B // 4, D), lambda r: (r, 0)),
            pl.BlockSpec((1, 1, D), lambda r, i=i: (i, 0, 0)),
            pl.BlockSpec((1, D, D), lambda r, i=i: (i, 0, 0)),
            pl.BlockSpec((1, D, D), lambda r, i=i: (i, 0, 0)),
        ],
        out_specs=pl.BlockSpec((B // 4, D), lambda r: (r, 0)),
        out_shape=jax.ShapeDtypeStruct((B, D), _F32),
        interpret=_INTERPRET,
    )(h, ln1, wv, wo)


def _att_route_body(h_ref, ln1_ref, wv_ref, wo_ref, ln2_ref, rw_ref,
                    hout_ref, n2_ref, log_ref):
    h = h_ref[...]
    n = _rms(h, ln1_ref[0, 0, :])
    hn = h + _dot(_dot(n, wv_ref[0]), wo_ref[0])
    hout_ref[...] = hn
    n2 = _rms(hn, ln2_ref[0, 0, :])
    n2_ref[...] = n2
    log_ref[...] = _dot(n2, rw_ref[...])


def _att_route(h, ln1, wv, wo, ln2, rw_pad, i):
    return pl.pallas_call(
        _att_route_body,
        grid=(4,),
        in_specs=[
            pl.BlockSpec((B // 4, D), lambda r: (r, 0)),
            pl.BlockSpec((1, 1, D), lambda r, i=i: (i, 0, 0)),
            pl.BlockSpec((1, D, D), lambda r, i=i: (i, 0, 0)),
            pl.BlockSpec((1, D, D), lambda r, i=i: (i, 0, 0)),
            pl.BlockSpec((1, 1, D), lambda r, i=i: (i, 0, 0)),
            pl.BlockSpec((D, 128), lambda r: (0, 0)),
        ],
        out_specs=[
            pl.BlockSpec((B // 4, D), lambda r: (r, 0)),
            pl.BlockSpec((B // 4, D), lambda r: (r, 0)),
            pl.BlockSpec((B // 4, 128), lambda r: (r, 0)),
        ],
        out_shape=[
            jax.ShapeDtypeStruct((B, D), _F32),
            jax.ShapeDtypeStruct((B, D), _F32),
            jax.ShapeDtypeStruct((B, 128), _F32),
        ],
        interpret=_INTERPRET,
    )(h, ln1, wv, wo, ln2, rw_pad)


def _route_body(log_ref, slot_ref, gate_ref):
    l = log_ref[...]
    lane = lax.broadcasted_iota(jnp.int32, (B, 128), 1)
    valid = lane < E
    lm = jnp.where(valid, l, -1e30)
    m = jnp.max(lm, axis=-1, keepdims=True)
    p = jnp.where(valid, jnp.exp(lm - m), 0.0)
    probs = p / jnp.sum(p, axis=-1, keepdims=True)
    gate = jnp.max(probs, axis=-1, keepdims=True)
    hit = (probs == gate) & valid
    idx = jnp.min(jnp.where(hit, lane, 10**6), axis=-1, keepdims=True)
    onehot = jnp.where(lane == idx, 1.0, 0.0).astype(_F32)
    r = lax.broadcasted_iota(jnp.int32, (B, B), 0)
    c = lax.broadcasted_iota(jnp.int32, (B, B), 1)
    tri = (c <= r).astype(_F32)
    counts = _dot(tri, onehot)
    pos = jnp.sum(jnp.where(lane == idx, counts, 0.0), axis=-1,
                  keepdims=True) - 1.0
    kept = pos < float(CAP)
    slot = jnp.where(kept, idx.astype(_F32) * CAP + pos, 1e7)
    slot_ref[...] = jnp.broadcast_to(slot, (B, 128))
    gate_ref[...] = jnp.broadcast_to(gate, (B, 128))


def _route(logits):
    return pl.pallas_call(
        _route_body,
        out_shape=[
            jax.ShapeDtypeStruct((B, 128), _F32),
            jax.ShapeDtypeStruct((B, 128), _F32),
        ],
        interpret=_INTERPRET,
    )(logits)


def _dispatch_body(slot_ref, n2_ref, ein_ref):
    slot = slot_ref[...][:, 0:1].astype(jnp.int32)
    lane = lax.broadcasted_iota(jnp.int32, (B, E * CAP), 1)
    mt = (jnp.broadcast_to(slot, (B, E * CAP)) == lane).astype(_F32)
    full = lax.dot_general(n2_ref[...], mt, (((0,), (0,)), ((), ())),
                           preferred_element_type=_F32)
    for e in range(E):
        ein_ref[e] = full[:, e * CAP:(e + 1) * CAP]


def _dispatch(slot_b, n2):
    return pl.pallas_call(
        _dispatch_body,
        in_specs=[
            pl.BlockSpec((B, 128), lambda: (0, 0)),
            pl.BlockSpec((B, D), lambda: (0, 0)),
        ],
        out_specs=pl.BlockSpec((E, D, CAP), lambda: (0, 0, 0)),
        out_shape=jax.ShapeDtypeStruct((E, D, CAP), _F32),
        interpret=_INTERPRET,
    )(slot_b, n2)


def _expert_body(eint_ref, wi_ref, wo_ref, out_ref):
    eint = eint_ref[0]
    h1_t = jnp.maximum(lax.dot_general(
        wi_ref[0, 0], eint, (((0,), (0,)), ((), ())),
        preferred_element_type=_F32), 0.0)
    out_ref[0] = lax.dot_general(
        wo_ref[0, 0], h1_t, (((0,), (0,)), ((), ())),
        preferred_element_type=_F32)


def _experts(ein_t, wi, wo, j):
    return pl.pallas_call(
        _expert_body,
        grid=(E,),
        in_specs=[
            pl.BlockSpec((1, D, CAP), lambda e: (e, 0, 0)),
            pl.BlockSpec((1, 1, D, FF), lambda e, j=j: (j, e, 0, 0)),
            pl.BlockSpec((1, 1, FF, D), lambda e, j=j: (j, e, 0, 0)),
        ],
        out_specs=pl.BlockSpec((1, D, CAP), lambda e: (e, 0, 0)),
        out_shape=jax.ShapeDtypeStruct((E, D, CAP), _F32),
        interpret=_INTERPRET,
    )(ein_t, wi, wo)


def _combine_body(slot_ref, gate_ref, h2_ref, h_ref, out_ref):
    slot = slot_ref[...][:, 0:1].astype(jnp.int32)
    lane = lax.broadcasted_iota(jnp.int32, (B, CAP), 1)
    gate = gate_ref[...][:, 0:1]
    y = None
    for e in range(E):
        ne = (jnp.broadcast_to(slot - e * CAP, (B, CAP))
              == lane).astype(_F32)
        part = lax.dot_general(ne, h2_ref[e], (((1,), (1,)), ((), ())),
                               preferred_element_type=_F32)
        y = part if y is None else y + part
    out_ref[...] = h_ref[...] + jnp.broadcast_to(gate, (B, D)) * y


def _combine(slot_b, gate_b, h2_t, h):
    return pl.pallas_call(
        _combine_body,
        in_specs=[
            pl.BlockSpec((B, 128), lambda: (0, 0)),
            pl.BlockSpec((B, 128), lambda: (0, 0)),
            pl.BlockSpec((E, D, CAP), lambda: (0, 0, 0)),
            pl.BlockSpec((B, D), lambda: (0, 0)),
        ],
        out_specs=pl.BlockSpec((B, D), lambda: (0, 0)),
        out_shape=jax.ShapeDtypeStruct((B, D), _F32),
        interpret=_INTERPRET,
    )(slot_b, gate_b, h2_t, h)


def _ffn_body(h_ref, ln2_ref, wi_ref, wo_ref, out_ref):
    f = pl.program_id(0)
    h = h_ref[...]
    n2 = _rms(h, ln2_ref[0, 0, :])
    h1 = jnp.maximum(_dot(n2, wi_ref[0]), 0.0)
    part = _dot(h1, wo_ref[0])

    @pl.when(f == 0)
    def _():
        out_ref[...] = h + part

    @pl.when(f != 0)
    def _():
        out_ref[...] += part


def _ffn(h, ln2, wi, wo, i, j):
    fb = FF // 2
    return pl.pallas_call(
        _ffn_body,
        grid=(2,),
        in_specs=[
            pl.BlockSpec((B, D), lambda f: (0, 0)),
            pl.BlockSpec((1, 1, D), lambda f, i=i: (i, 0, 0)),
            pl.BlockSpec((1, D, fb), lambda f, j=j: (j, 0, f)),
            pl.BlockSpec((1, fb, D), lambda f, j=j: (j, f, 0)),
        ],
        out_specs=pl.BlockSpec((B, D), lambda f: (0, 0)),
        out_shape=jax.ShapeDtypeStruct((B, D), _F32),
        interpret=_INTERPRET,
    )(h, ln2, wi, wo)


def _final_body(h_ref, ln_ref, w_ref, b_ref, out_ref):
    n = _rms(h_ref[...], ln_ref[0, :])
    out_ref[...] = _dot(n, w_ref[...]) + b_ref[0, :]


def _final(h, final_ln, fcw_pad, fcb_pad):
    return pl.pallas_call(
        _final_body,
        out_shape=jax.ShapeDtypeStruct((B, 128), _F32),
        interpret=_INTERPRET,
    )(h, final_ln, fcw_pad, fcb_pad)


def kernel(x, proj_W, proj_b, attn_q, attn_k, attn_v, attn_o, ln1, ln2,
           router_W, moe_wi, moe_wo, ffn_wi, ffn_wo, final_ln, fc_W, fc_b):
    del attn_q, attn_k
    x2 = x.reshape(B, -1)
    ln1 = ln1.reshape(L, 1, D)
    ln2 = ln2.reshape(L, 1, D)
    h = _proj(x2, proj_W, proj_b)
    rw_pad = jnp.pad(router_W, ((0, 0), (0, 0), (0, 128 - E)))
    for i in range(L):
        j = i // 2
        if i % 2 == 0:
            h = _att(h, ln1, attn_v, attn_o, i)
            h = _ffn(h, ln2, ffn_wi, ffn_wo, i, j)
        else:
            h, n2, logits = _att_route(h, ln1, attn_v, attn_o, ln2,
                                       rw_pad[j], i)
            slot_b, gate_b = _route(logits)
            ein_t = _dispatch(slot_b, n2)
            h2_t = _experts(ein_t, moe_wi, moe_wo, j)
            h = _combine(slot_b, gate_b, h2_t, h)
    out = _final(h, final_ln.reshape(1, D),
                 jnp.pad(fc_W, ((0, 0), (0, 128 - 10))),
                 jnp.pad(fc_b, (0, 128 - 10)).reshape(1, 128))
    return out[:, :10]


# speedup vs baseline: 2.1088x; 1.0525x over previous
import jax
import jax.numpy as jnp
from jax import lax
from jax.experimental import pallas as pl

B = 1024
D = 768
FF = 3072
E = 8
L = 6
CAP = 160

_F32 = jnp.float32
_INTERPRET = False


def _rms(h, w):
    return h * lax.rsqrt(jnp.mean(h * h, axis=-1, keepdims=True) + 1e-6) * w


def _dot(a, b):
    return jnp.dot(a, b, preferred_element_type=_F32)


def _proj_body(x_ref, w_ref, b_ref, out_ref):
    out_ref[...] = _dot(x_ref[...], w_ref[...]) + b_ref[0, :]


def _proj(x2, w, b):
    return pl.pallas_call(
        _proj_body,
        grid=(4,),
        in_specs=[
            pl.BlockSpec((B // 4, 3072), lambda i: (i, 0)),
            pl.BlockSpec((3072, D), lambda i: (0, 0)),
            pl.BlockSpec((1, D), lambda i: (0, 0)),
        ],
        out_specs=pl.BlockSpec((B // 4, D), lambda i: (i, 0)),
        out_shape=jax.ShapeDtypeStruct((B, D), _F32),
        interpret=_INTERPRET,
    )(x2, w, b.reshape(1, D))


def _att_body(h_ref, ln1_ref, wv_ref, wo_ref, out_ref):
    h = h_ref[...]
    n = _rms(h, ln1_ref[0, 0, :])
    out_ref[...] = h + _dot(_dot(n, wv_ref[0]), wo_ref[0])


def _att(h, ln1, wv, wo, i):
    return pl.pallas_call(
        _att_body,
        grid=(4,),
        in_specs=[
            pl.BlockSpec((B // 4, D), lambda r: (r, 0)),
            pl.BlockSpec((1, 1, D), lambda r, i=i: (i, 0, 0)),
            pl.BlockSpec((1, D, D), lambda r, i=i: (i, 0, 0)),
            pl.BlockSpec((1, D, D), lambda r, i=i: (i, 0, 0)),
        ],
        out_specs=pl.BlockSpec((B // 4, D), lambda r: (r, 0)),
        out_shape=jax.ShapeDtypeStruct((B, D), _F32),
        interpret=_INTERPRET,
    )(h, ln1, wv, wo)


def _att_route_body(h_ref, ln1_ref, wv_ref, wo_ref, ln2_ref, rw_ref,
                    hout_ref, n2_ref, log_ref):
    h = h_ref[...]
    n = _rms(h, ln1_ref[0, 0, :])
    hn = h + _dot(_dot(n, wv_ref[0]), wo_ref[0])
    hout_ref[...] = hn
    n2 = _rms(hn, ln2_ref[0, 0, :])
    n2_ref[...] = n2
    log_ref[...] = _dot(n2, rw_ref[...])


def _att_route(h, ln1, wv, wo, ln2, rw_pad, i):
    return pl.pallas_call(
        _att_route_body,
        grid=(4,),
        in_specs=[
            pl.BlockSpec((B // 4, D), lambda r: (r, 0)),
            pl.BlockSpec((1, 1, D), lambda r, i=i: (i, 0, 0)),
            pl.BlockSpec((1, D, D), lambda r, i=i: (i, 0, 0)),
            pl.BlockSpec((1, D, D), lambda r, i=i: (i, 0, 0)),
            pl.BlockSpec((1, 1, D), lambda r, i=i: (i, 0, 0)),
            pl.BlockSpec((D, 128), lambda r: (0, 0)),
        ],
        out_specs=[
            pl.BlockSpec((B // 4, D), lambda r: (r, 0)),
            pl.BlockSpec((B // 4, D), lambda r: (r, 0)),
            pl.BlockSpec((B // 4, 128), lambda r: (r, 0)),
        ],
        out_shape=[
            jax.ShapeDtypeStruct((B, D), _F32),
            jax.ShapeDtypeStruct((B, D), _F32),
            jax.ShapeDtypeStruct((B, 128), _F32),
        ],
        interpret=_INTERPRET,
    )(h, ln1, wv, wo, ln2, rw_pad)


def _route_body(log_ref, slot_ref, gate_ref):
    l = log_ref[...]
    lane = lax.broadcasted_iota(jnp.int32, (B, 128), 1)
    valid = lane < E
    lm = jnp.where(valid, l, -1e30)
    m = jnp.max(lm, axis=-1, keepdims=True)
    p = jnp.where(valid, jnp.exp(lm - m), 0.0)
    probs = p / jnp.sum(p, axis=-1, keepdims=True)
    gate = jnp.max(probs, axis=-1, keepdims=True)
    hit = (probs == gate) & valid
    idx = jnp.min(jnp.where(hit, lane, 10**6), axis=-1, keepdims=True)
    onehot = jnp.where(lane == idx, 1.0, 0.0).astype(_F32)
    r = lax.broadcasted_iota(jnp.int32, (B, B), 0)
    c = lax.broadcasted_iota(jnp.int32, (B, B), 1)
    tri = (c <= r).astype(_F32)
    counts = _dot(tri, onehot)
    pos = jnp.sum(jnp.where(lane == idx, counts, 0.0), axis=-1,
                  keepdims=True) - 1.0
    kept = pos < float(CAP)
    slot = jnp.where(kept, idx.astype(_F32) * CAP + pos, 1e7)
    slot_ref[...] = jnp.broadcast_to(slot, (B, 128))
    gate_ref[...] = jnp.broadcast_to(gate, (B, 128))


def _route(logits):
    return pl.pallas_call(
        _route_body,
        out_shape=[
            jax.ShapeDtypeStruct((B, 128), _F32),
            jax.ShapeDtypeStruct((B, 128), _F32),
        ],
        interpret=_INTERPRET,
    )(logits)


def _dispatch_body(slot_ref, n2_ref, ein_ref):
    slot = slot_ref[...][:, 0:1].astype(jnp.int32)
    lane = lax.broadcasted_iota(jnp.int32, (B, E * CAP), 1)
    mt = (jnp.broadcast_to(slot, (B, E * CAP)) == lane).astype(_F32)
    full = lax.dot_general(n2_ref[...], mt, (((0,), (0,)), ((), ())),
                           preferred_element_type=_F32)
    for e in range(E):
        ein_ref[e] = full[:, e * CAP:(e + 1) * CAP]


def _dispatch(slot_b, n2):
    return pl.pallas_call(
        _dispatch_body,
        in_specs=[
            pl.BlockSpec((B, 128), lambda: (0, 0)),
            pl.BlockSpec((B, D), lambda: (0, 0)),
        ],
        out_specs=pl.BlockSpec((E, D, CAP), lambda: (0, 0, 0)),
        out_shape=jax.ShapeDtypeStruct((E, D, CAP), _F32),
        interpret=_INTERPRET,
    )(slot_b, n2)


def _expert_body(eint_ref, wi_ref, wo_ref, out_ref):
    eint = eint_ref[0]
    h1_t = jnp.maximum(lax.dot_general(
        wi_ref[0, 0], eint, (((0,), (0,)), ((), ())),
        preferred_element_type=_F32), 0.0)
    out_ref[0] = lax.dot_general(
        wo_ref[0, 0], h1_t, (((0,), (0,)), ((), ())),
        preferred_element_type=_F32)


def _experts(ein_t, wi, wo, j):
    return pl.pallas_call(
        _expert_body,
        grid=(E,),
        in_specs=[
            pl.BlockSpec((1, D, CAP), lambda e: (e, 0, 0)),
            pl.BlockSpec((1, 1, D, FF), lambda e, j=j: (j, e, 0, 0)),
            pl.BlockSpec((1, 1, FF, D), lambda e, j=j: (j, e, 0, 0)),
        ],
        out_specs=pl.BlockSpec((1, D, CAP), lambda e: (e, 0, 0)),
        out_shape=jax.ShapeDtypeStruct((E, D, CAP), _F32),
        interpret=_INTERPRET,
    )(ein_t, wi, wo)


def _combine_body(slot_ref, gate_ref, h2_ref, h_ref, out_ref):
    slot = slot_ref[...][:, 0:1].astype(jnp.int32)
    lane = lax.broadcasted_iota(jnp.int32, (B, CAP), 1)
    gate = gate_ref[...][:, 0:1]
    y = None
    for e in range(E):
        ne = (jnp.broadcast_to(slot - e * CAP, (B, CAP))
              == lane).astype(_F32)
        part = lax.dot_general(ne, h2_ref[e], (((1,), (1,)), ((), ())),
                               preferred_element_type=_F32)
        y = part if y is None else y + part
    out_ref[...] = h_ref[...] + jnp.broadcast_to(gate, (B, D)) * y


def _combine(slot_b, gate_b, h2_t, h):
    return pl.pallas_call(
        _combine_body,
        in_specs=[
            pl.BlockSpec((B, 128), lambda: (0, 0)),
            pl.BlockSpec((B, 128), lambda: (0, 0)),
            pl.BlockSpec((E, D, CAP), lambda: (0, 0, 0)),
            pl.BlockSpec((B, D), lambda: (0, 0)),
        ],
        out_specs=pl.BlockSpec((B, D), lambda: (0, 0)),
        out_shape=jax.ShapeDtypeStruct((B, D), _F32),
        interpret=_INTERPRET,
    )(slot_b, gate_b, h2_t, h)


def _ffn_body(h_ref, ln2_ref, wi_ref, wo_ref, out_ref):
    f = pl.program_id(0)
    h = h_ref[...]
    n2 = _rms(h, ln2_ref[0, 0, :])
    h1 = jnp.maximum(_dot(n2, wi_ref[0]), 0.0)
    part = _dot(h1, wo_ref[0])

    @pl.when(f == 0)
    def _():
        out_ref[...] = h + part

    @pl.when(f != 0)
    def _():
        out_ref[...] += part


def _ffn(h, ln2, wi, wo, i, j):
    fb = FF // 2
    return pl.pallas_call(
        _ffn_body,
        grid=(2,),
        in_specs=[
            pl.BlockSpec((B, D), lambda f: (0, 0)),
            pl.BlockSpec((1, 1, D), lambda f, i=i: (i, 0, 0)),
            pl.BlockSpec((1, D, fb), lambda f, j=j: (j, 0, f)),
            pl.BlockSpec((1, fb, D), lambda f, j=j: (j, f, 0)),
        ],
        out_specs=pl.BlockSpec((B, D), lambda f: (0, 0)),
        out_shape=jax.ShapeDtypeStruct((B, D), _F32),
        interpret=_INTERPRET,
    )(h, ln2, wi, wo)


def _final_body(h_ref, ln_ref, w_ref, b_ref, out_ref):
    n = _rms(h_ref[...], ln_ref[0, :])
    out_ref[...] = _dot(n, w_ref[...]) + b_ref[0, :]


def _final(h, final_ln, fcw_pad, fcb_pad):
    return pl.pallas_call(
        _final_body,
        out_shape=jax.ShapeDtypeStruct((B, 128), _F32),
        interpret=_INTERPRET,
    )(h, final_ln, fcw_pad, fcb_pad)


def _moe_body(log_ref, n2_ref, h_ref, wi_ref, wo_ref, out_ref,
              slot_s, gate_s, ein_s, h2_s, y_s):
    e = pl.program_id(0)
    f = pl.program_id(1)

    @pl.when(jnp.logical_and(e == 0, f == 0))
    def _():
        l = log_ref[...]
        lane = lax.broadcasted_iota(jnp.int32, (B, 128), 1)
        valid = lane < E
        lm = jnp.where(valid, l, -1e30)
        m = jnp.max(lm, axis=-1, keepdims=True)
        p = jnp.where(valid, jnp.exp(lm - m), 0.0)
        probs = p / jnp.sum(p, axis=-1, keepdims=True)
        gate = jnp.max(probs, axis=-1, keepdims=True)
        hit = (probs == gate) & valid
        idx = jnp.min(jnp.where(hit, lane, 10**6), axis=-1, keepdims=True)
        onehot = jnp.where(lane == idx, 1.0, 0.0).astype(_F32)
        r = lax.broadcasted_iota(jnp.int32, (B, B), 0)
        c = lax.broadcasted_iota(jnp.int32, (B, B), 1)
        tri = (c <= r).astype(_F32)
        counts = _dot(tri, onehot)
        pos = jnp.sum(jnp.where(lane == idx, counts, 0.0), axis=-1,
                      keepdims=True) - 1.0
        kept = pos < float(CAP)
        slot = jnp.where(kept, idx.astype(_F32) * CAP + pos, 1e7)
        slot_s[...] = jnp.broadcast_to(slot, (B, 128))
        gate_s[...] = jnp.broadcast_to(gate, (B, 128))

    slot = slot_s[...][:, 0:1].astype(jnp.int32)
    lane = lax.broadcasted_iota(jnp.int32, (B, CAP), 1) + e * CAP
    mt = (jnp.broadcast_to(slot, (B, CAP)) == lane).astype(_F32)

    @pl.when(f == 0)
    def _():
        ein_s[...] = lax.dot_general(
            n2_ref[...], mt, (((0,), (0,)), ((), ())),
            preferred_element_type=_F32)

    h1_t = jnp.maximum(lax.dot_general(
        wi_ref[0, 0], ein_s[...], (((0,), (0,)), ((), ())),
        preferred_element_type=_F32), 0.0)
    h2p = lax.dot_general(wo_ref[0, 0], h1_t, (((0,), (0,)), ((), ())),
                          preferred_element_type=_F32)

    @pl.when(f == 0)
    def _():
        h2_s[...] = h2p

    @pl.when(f == 1)
    def _():
        h2 = h2_s[...] + h2p
        part = lax.dot_general(mt, h2, (((1,), (1,)), ((), ())),
                               preferred_element_type=_F32)

        @pl.when(e == 0)
        def _():
            y_s[...] = part

        @pl.when(e != 0)
        def _():
            y_s[...] += part

        @pl.when(e == E - 1)
        def _():
            gate = gate_s[...][:, 0:1]
            out_ref[...] = (h_ref[...]
                            + jnp.broadcast_to(gate, (B, D)) * y_s[...])


def _moe(logits, n2, h, wi, wo, j):
    fb = FF // 2
    from jax.experimental.pallas import tpu as pltpu
    return pl.pallas_call(
        _moe_body,
        grid=(E, 2),
        in_specs=[
            pl.BlockSpec((B, 128), lambda e, f: (0, 0)),
            pl.BlockSpec((B, D), lambda e, f: (0, 0)),
            pl.BlockSpec((B, D), lambda e, f: (0, 0)),
            pl.BlockSpec((1, 1, D, fb), lambda e, f, j=j: (j, e, 0, f)),
            pl.BlockSpec((1, 1, fb, D), lambda e, f, j=j: (j, e, f, 0)),
        ],
        out_specs=pl.BlockSpec((B, D), lambda e, f: (0, 0)),
        out_shape=jax.ShapeDtypeStruct((B, D), _F32),
        scratch_shapes=[
            pltpu.VMEM((B, 128), _F32),
            pltpu.VMEM((B, 128), _F32),
            pltpu.VMEM((D, CAP), _F32),
            pltpu.VMEM((D, CAP), _F32),
            pltpu.VMEM((B, D), _F32),
        ],
        interpret=_INTERPRET,
    )(logits, n2, h, wi, wo)


def kernel(x, proj_W, proj_b, attn_q, attn_k, attn_v, attn_o, ln1, ln2,
           router_W, moe_wi, moe_wo, ffn_wi, ffn_wo, final_ln, fc_W, fc_b):
    del attn_q, attn_k
    x2 = x.reshape(B, -1)
    ln1 = ln1.reshape(L, 1, D)
    ln2 = ln2.reshape(L, 1, D)
    h = _proj(x2, proj_W, proj_b)
    rw_pad = jnp.pad(router_W, ((0, 0), (0, 0), (0, 128 - E)))
    for i in range(L):
        j = i // 2
        if i % 2 == 0:
            h = _att(h, ln1, attn_v, attn_o, i)
            h = _ffn(h, ln2, ffn_wi, ffn_wo, i, j)
        else:
            h, n2, logits = _att_route(h, ln1, attn_v, attn_o, ln2,
                                       rw_pad[j], i)
            h = _moe(logits, n2, h, moe_wi, moe_wo, j)
    out = _final(h, final_ln.reshape(1, D),
                 jnp.pad(fc_W, ((0, 0), (0, 128 - 10))),
                 jnp.pad(fc_b, (0, 128 - 10)).reshape(1, 128))
    return out[:, :10]


# speedup vs baseline: 2.1188x; 1.0047x over previous
import jax
import jax.numpy as jnp
from jax import lax
from jax.experimental import pallas as pl
from jax.experimental.pallas import tpu as pltpu

B = 1024
D = 768
FF = 3072
E = 8
L = 6
CAP = 160

_F32 = jnp.float32


def _rms(h, w):
    return h * lax.rsqrt(jnp.mean(h * h, axis=-1, keepdims=True) + 1e-6) * w


def _dot(a, b):
    return jnp.dot(a, b, preferred_element_type=_F32)


def _proj_body(x_ref, w_ref, b_ref, out_ref):
    out_ref[...] = _dot(x_ref[...], w_ref[...]) + b_ref[0, :]


def _proj(x2, w, b):
    return pl.pallas_call(
        _proj_body,
        grid=(4,),
        in_specs=[
            pl.BlockSpec((B // 4, 3072), lambda i: (i, 0)),
            pl.BlockSpec((3072, D), lambda i: (0, 0)),
            pl.BlockSpec((1, D), lambda i: (0, 0)),
        ],
        out_specs=pl.BlockSpec((B // 4, D), lambda i: (i, 0)),
        out_shape=jax.ShapeDtypeStruct((B, D), _F32),
    )(x2, w, b.reshape(1, D))


def _att_body(h_ref, ln1_ref, wv_ref, wo_ref, out_ref):
    h = h_ref[...]
    n = _rms(h, ln1_ref[0, 0, :])
    out_ref[...] = h + _dot(_dot(n, wv_ref[0]), wo_ref[0])


def _att(h, ln1, wv, wo, i):
    return pl.pallas_call(
        _att_body,
        grid=(4,),
        in_specs=[
            pl.BlockSpec((B // 4, D), lambda r: (r, 0)),
            pl.BlockSpec((1, 1, D), lambda r, i=i: (i, 0, 0)),
            pl.BlockSpec((1, D, D), lambda r, i=i: (i, 0, 0)),
            pl.BlockSpec((1, D, D), lambda r, i=i: (i, 0, 0)),
        ],
        out_specs=pl.BlockSpec((B // 4, D), lambda r: (r, 0)),
        out_shape=jax.ShapeDtypeStruct((B, D), _F32),
    )(h, ln1, wv, wo)


def _att_route_body(h_ref, ln1_ref, wv_ref, wo_ref, ln2_ref, rw_ref,
                    hout_ref, n2_ref, log_ref):
    h = h_ref[...]
    n = _rms(h, ln1_ref[0, 0, :])
    hn = h + _dot(_dot(n, wv_ref[0]), wo_ref[0])
    hout_ref[...] = hn
    n2 = _rms(hn, ln2_ref[0, 0, :])
    n2_ref[...] = n2
    log_ref[...] = _dot(n2, rw_ref[...])


def _att_route(h, ln1, wv, wo, ln2, rw_pad, i):
    return pl.pallas_call(
        _att_route_body,
        grid=(4,),
        in_specs=[
            pl.BlockSpec((B // 4, D), lambda r: (r, 0)),
            pl.BlockSpec((1, 1, D), lambda r, i=i: (i, 0, 0)),
            pl.BlockSpec((1, D, D), lambda r, i=i: (i, 0, 0)),
            pl.BlockSpec((1, D, D), lambda r, i=i: (i, 0, 0)),
            pl.BlockSpec((1, 1, D), lambda r, i=i: (i, 0, 0)),
            pl.BlockSpec((D, 128), lambda r: (0, 0)),
        ],
        out_specs=[
            pl.BlockSpec((B // 4, D), lambda r: (r, 0)),
            pl.BlockSpec((B // 4, D), lambda r: (r, 0)),
            pl.BlockSpec((B // 4, 128), lambda r: (r, 0)),
        ],
        out_shape=[
            jax.ShapeDtypeStruct((B, D), _F32),
            jax.ShapeDtypeStruct((B, D), _F32),
            jax.ShapeDtypeStruct((B, 128), _F32),
        ],
    )(h, ln1, wv, wo, ln2, rw_pad)


def _ffn_body(h_ref, ln2_ref, wi_ref, wo_ref, out_ref):
    f = pl.program_id(0)
    h = h_ref[...]
    n2 = _rms(h, ln2_ref[0, 0, :])
    h1 = jnp.maximum(_dot(n2, wi_ref[0]), 0.0)
    part = _dot(h1, wo_ref[0])

    @pl.when(f == 0)
    def _():
        out_ref[...] = h + part

    @pl.when(f != 0)
    def _():
        out_ref[...] += part


def _ffn(h, ln2, wi, wo, i, j):
    fb = FF // 2
    return pl.pallas_call(
        _ffn_body,
        grid=(2,),
        in_specs=[
            pl.BlockSpec((B, D), lambda f: (0, 0)),
            pl.BlockSpec((1, 1, D), lambda f, i=i: (i, 0, 0)),
            pl.BlockSpec((1, D, fb), lambda f, j=j: (j, 0, f)),
            pl.BlockSpec((1, fb, D), lambda f, j=j: (j, f, 0)),
        ],
        out_specs=pl.BlockSpec((B, D), lambda f: (0, 0)),
        out_shape=jax.ShapeDtypeStruct((B, D), _F32),
    )(h, ln2, wi, wo)


def _final_body(h_ref, ln_ref, w_ref, b_ref, out_ref):
    n = _rms(h_ref[...], ln_ref[0, :])
    out_ref[...] = _dot(n, w_ref[...]) + b_ref[0, :]


def _final(h, final_ln, fcw_pad, fcb_pad):
    return pl.pallas_call(
        _final_body,
        out_shape=jax.ShapeDtypeStruct((B, 128), _F32),
    )(h, final_ln, fcw_pad, fcb_pad)


def _moe_body(log_ref, n2_ref, h_ref, wi_ref, wo_ref, out_ref,
              slot_s, gate_s, ein_s, h2_s, y_s):
    e = pl.program_id(0)
    f = pl.program_id(1)

    @pl.when(jnp.logical_and(e == 0, f == 0))
    def _():
        l = log_ref[...]
        lane = lax.broadcasted_iota(jnp.int32, (B, 128), 1)
        valid = lane < E
        lm = jnp.where(valid, l, -1e30)
        m = jnp.max(lm, axis=-1, keepdims=True)
        p = jnp.where(valid, jnp.exp(lm - m), 0.0)
        probs = p / jnp.sum(p, axis=-1, keepdims=True)
        gate = jnp.max(probs, axis=-1, keepdims=True)
        hit = (probs == gate) & valid
        idx = jnp.min(jnp.where(hit, lane, 10**6), axis=-1, keepdims=True)
        onehot = jnp.where(lane == idx, 1.0, 0.0).astype(_F32)
        r = lax.broadcasted_iota(jnp.int32, (B, B), 0)
        c = lax.broadcasted_iota(jnp.int32, (B, B), 1)
        tri = (c <= r).astype(_F32)
        counts = _dot(tri, onehot)
        pos = jnp.sum(jnp.where(lane == idx, counts, 0.0), axis=-1,
                      keepdims=True) - 1.0
        kept = pos < float(CAP)
        slot = jnp.where(kept, idx.astype(_F32) * CAP + pos, 1e7)
        slot_s[...] = jnp.broadcast_to(slot, (B, 128))
        gate_s[...] = jnp.broadcast_to(gate, (B, 128))

    slot = slot_s[...][:, 0:1].astype(jnp.int32)
    lane = lax.broadcasted_iota(jnp.int32, (B, CAP), 1) + e * CAP
    mt = (jnp.broadcast_to(slot, (B, CAP)) == lane).astype(_F32)

    @pl.when(f == 0)
    def _():
        ein_s[...] = lax.dot_general(
            n2_ref[...], mt, (((0,), (0,)), ((), ())),
            preferred_element_type=_F32)

    h1_t = jnp.maximum(lax.dot_general(
        wi_ref[0, 0], ein_s[...], (((0,), (0,)), ((), ())),
        preferred_element_type=_F32), 0.0)
    h2p = lax.dot_general(wo_ref[0, 0], h1_t, (((0,), (0,)), ((), ())),
                          preferred_element_type=_F32)

    @pl.when(f == 0)
    def _():
        h2_s[...] = h2p

    @pl.when(f == 1)
    def _():
        h2 = h2_s[...] + h2p
        part = lax.dot_general(mt, h2, (((1,), (1,)), ((), ())),
                               preferred_element_type=_F32)

        @pl.when(e == 0)
        def _():
            y_s[...] = part

        @pl.when(e != 0)
        def _():
            y_s[...] += part

        @pl.when(e == E - 1)
        def _():
            gate = gate_s[...][:, 0:1]
            out_ref[...] = (h_ref[...]
                            + jnp.broadcast_to(gate, (B, D)) * y_s[...])


def _moe(logits, n2, h, wi, wo, j):
    fb = FF // 2
    return pl.pallas_call(
        _moe_body,
        grid=(E, 2),
        in_specs=[
            pl.BlockSpec((B, 128), lambda e, f: (0, 0)),
            pl.BlockSpec((B, D), lambda e, f: (0, 0)),
            pl.BlockSpec((B, D), lambda e, f: (0, 0)),
            pl.BlockSpec((1, 1, D, fb), lambda e, f, j=j: (j, e, 0, f)),
            pl.BlockSpec((1, 1, fb, D), lambda e, f, j=j: (j, e, f, 0)),
        ],
        out_specs=pl.BlockSpec((B, D), lambda e, f: (0, 0)),
        out_shape=jax.ShapeDtypeStruct((B, D), _F32),
        scratch_shapes=[
            pltpu.VMEM((B, 128), _F32),
            pltpu.VMEM((B, 128), _F32),
            pltpu.VMEM((D, CAP), _F32),
            pltpu.VMEM((D, CAP), _F32),
            pltpu.VMEM((B, D), _F32),
        ],
    )(logits, n2, h, wi, wo)


def kernel(x, proj_W, proj_b, attn_q, attn_k, attn_v, attn_o, ln1, ln2,
           router_W, moe_wi, moe_wo, ffn_wi, ffn_wo, final_ln, fc_W, fc_b):
    del attn_q, attn_k
    x2 = x.reshape(B, -1)
    ln1 = ln1.reshape(L, 1, D)
    ln2 = ln2.reshape(L, 1, D)
    h = _proj(x2, proj_W, proj_b)
    rw_pad = jnp.pad(router_W, ((0, 0), (0, 0), (0, 128 - E)))
    for i in range(L):
        j = i // 2
        if i % 2 == 0:
            h = _att(h, ln1, attn_v, attn_o, i)
            h = _ffn(h, ln2, ffn_wi, ffn_wo, i, j)
        else:
            h, n2, logits = _att_route(h, ln1, attn_v, attn_o, ln2,
                                       rw_pad[j], i)
            h = _moe(logits, n2, h, moe_wi, moe_wo, j)
    out = _final(h, final_ln.reshape(1, D),
                 jnp.pad(fc_W, ((0, 0), (0, 128 - 10))),
                 jnp.pad(fc_b, (0, 128 - 10)).reshape(1, 128))
    return out[:, :10]
